# baseline (speedup 1.0000x reference)
import jax
import jax.numpy as jnp
from jax.experimental import pallas as pl
from jax.experimental.pallas import tpu as pltpu

_THRESH = (50.0, 100.0, 200.0)
_HWB = 512
_TJ = 4


def _body(cls_ref, x_ref, t_ref, m0_ref, m1_ref, m2_ref, dmin_ref, pcls_ref,
          xt_ref, xn_ref, tt_ref):
    j = pl.program_id(1)
    n_t = pl.num_programs(1)

    @pl.when(j == 0)
    def _prep():
        xt = jnp.swapaxes(x_ref[...], 1, 2)
        xt_ref[...] = xt
        xn_ref[...] = jnp.sum(xt * xt, axis=1)

    tt_ref[...] = jnp.swapaxes(t_ref[...], 1, 2)
    tj, d_dim, hwb = tt_ref.shape
    b_dim = xt_ref.shape[0]
    n_ch = d_dim // 8
    bq = 4 if b_dim % 4 == 0 else b_dim
    n_q = b_dim // bq

    first_step = j == 0
    tn = [None] * tj
    for q in range(n_q):
        rows = slice(q * bq, (q + 1) * bq)
        accs = [jnp.zeros((bq, 8, hwb), jnp.float32) for _ in range(tj)]
        tn_accs = [jnp.zeros((8, hwb), jnp.float32) for _ in range(tj)]
        for k in range(n_ch):
            sl = slice(k * 8, (k + 1) * 8)
            xc = xt_ref[rows, sl, :]
            for u in range(tj):
                tc = tt_ref[u, sl, :]
                accs[u] = accs[u] + xc * tc[None]
                if q == 0:
                    tn_accs[u] = tn_accs[u] + tc * tc
        if q == 0:
            tn = [jnp.sum(a, axis=0) for a in tn_accs]
        xn = xn_ref[rows, :]
        for u in range(tj):
            cross = jnp.sum(accs[u], axis=1)
            dist = (xn + tn[u][None, :]) - 2.0 * cross
            cls = cls_ref[j * tj + u]
            prev = dmin_ref[rows, :]
            better = (dist < prev) | first_step if u == 0 else (dist < prev)
            dmin_ref[rows, :] = jnp.where(better, dist, prev)
            pcls_ref[rows, :] = jnp.where(better, cls, pcls_ref[rows, :])

    @pl.when(j == n_t - 1)
    def _masks():
        d = dmin_ref[...]
        m0_ref[...] = d <= _THRESH[0]
        m1_ref[...] = d <= _THRESH[1]
        m2_ref[...] = d <= _THRESH[2]


def kernel(frame_embeddings, templates, template_classes):
    B, HW, D = frame_embeddings.shape
    T = templates.shape[0]
    n_hw = HW // _HWB

    grid_spec = pltpu.PrefetchScalarGridSpec(
        num_scalar_prefetch=1,
        grid=(n_hw, T // _TJ),
        in_specs=[
            pl.BlockSpec((B, _HWB, D), lambda i, j, cls: (0, i, 0)),
            pl.BlockSpec((_TJ, _HWB, D), lambda i, j, cls: (j, i, 0)),
        ],
        out_specs=[
            pl.BlockSpec((B, _HWB), lambda i, j, cls: (0, i)) for _ in range(5)
        ],
        scratch_shapes=[
            pltpu.VMEM((B, D, _HWB), jnp.float32),
            pltpu.VMEM((B, _HWB), jnp.float32),
            pltpu.VMEM((_TJ, D, _HWB), jnp.float32),
        ],
    )
    out_shapes = (
        jax.ShapeDtypeStruct((B, HW), jnp.bool_),
        jax.ShapeDtypeStruct((B, HW), jnp.bool_),
        jax.ShapeDtypeStruct((B, HW), jnp.bool_),
        jax.ShapeDtypeStruct((B, HW), jnp.float32),
        jax.ShapeDtypeStruct((B, HW), jnp.int32),
    )
    m0, m1, m2, dmin, pcls = pl.pallas_call(
        _body,
        grid_spec=grid_spec,
        out_shape=out_shapes,
        compiler_params=pltpu.CompilerParams(
            dimension_semantics=("parallel", "arbitrary"),
        ),
    )(template_classes, frame_embeddings, templates)
    return m0, m1, m2, dmin, pcls


# speedup vs baseline: 1.3962x; 1.3962x over previous
import jax
import jax.numpy as jnp
from jax.experimental import pallas as pl
from jax.experimental.pallas import tpu as pltpu

_THRESH = (50.0, 100.0, 200.0)
_HWB = 512
_TJ = 4


def _body(cls_ref, x_ref, t_ref, m0_ref, m1_ref, m2_ref, dmin_ref, pcls_ref,
          xt_ref, xn_ref, tt_ref):
    j = pl.program_id(1)
    n_t = pl.num_programs(1)

    @pl.when(j == 0)
    def _prep():
        xt = jnp.swapaxes(x_ref[...], 1, 2)
        xt_ref[...] = xt
        xn_ref[...] = jnp.sum(xt * xt, axis=1)

    tt_ref[...] = jnp.swapaxes(t_ref[...], 1, 2)
    tj, d_dim, hwb = tt_ref.shape
    b_dim = xt_ref.shape[0]
    n_ch = d_dim // 8
    bq = 4 if b_dim % 4 == 0 else b_dim
    n_q = b_dim // bq

    tn_acc = jnp.zeros((tj, 8, hwb), jnp.float32)
    for k in range(n_ch):
        tc = tt_ref[:, k * 8:(k + 1) * 8, :]
        tn_acc = tn_acc + tc * tc
    tn = jnp.sum(tn_acc, axis=1)

    first_step = j == 0
    for q in range(n_q):
        rows = slice(q * bq, (q + 1) * bq)
        accs = [jnp.zeros((bq, 8, hwb), jnp.float32) for _ in range(tj)]
        for k in range(n_ch):
            sl = slice(k * 8, (k + 1) * 8)
            xc = xt_ref[rows, sl, :]
            for u in range(tj):
                accs[u] = accs[u] + xc * tt_ref[u, sl, :][None]
        xn = xn_ref[rows, :]
        for u in range(tj):
            cross = jnp.sum(accs[u], axis=1)
            dist = (xn + tn[u]) - 2.0 * cross
            cls = cls_ref[j * tj + u]
            if u == 0:
                @pl.when(first_step)
                def _init(dist=dist, cls=cls, rows=rows):
                    dmin_ref[rows, :] = dist
                    pcls_ref[rows, :] = jnp.full(dist.shape, cls, jnp.int32)

                @pl.when(jnp.logical_not(first_step))
                def _upd(dist=dist, cls=cls, rows=rows):
                    prev = dmin_ref[rows, :]
                    better = dist < prev
                    dmin_ref[rows, :] = jnp.where(better, dist, prev)
                    pcls_ref[rows, :] = jnp.where(better, cls,
                                                  pcls_ref[rows, :])
            else:
                prev = dmin_ref[rows, :]
                better = dist < prev
                dmin_ref[rows, :] = jnp.where(better, dist, prev)
                pcls_ref[rows, :] = jnp.where(better, cls, pcls_ref[rows, :])

    @pl.when(j == n_t - 1)
    def _masks():
        d = dmin_ref[...]
        m0_ref[...] = d <= _THRESH[0]
        m1_ref[...] = d <= _THRESH[1]
        m2_ref[...] = d <= _THRESH[2]


def kernel(frame_embeddings, templates, template_classes):
    B, HW, D = frame_embeddings.shape
    T = templates.shape[0]
    n_hw = HW // _HWB

    grid_spec = pltpu.PrefetchScalarGridSpec(
        num_scalar_prefetch=1,
        grid=(n_hw, T // _TJ),
        in_specs=[
            pl.BlockSpec((B, _HWB, D), lambda i, j, cls: (0, i, 0)),
            pl.BlockSpec((_TJ, _HWB, D), lambda i, j, cls: (j, i, 0)),
        ],
        out_specs=[
            pl.BlockSpec((B, _HWB), lambda i, j, cls: (0, i)) for _ in range(5)
        ],
        scratch_shapes=[
            pltpu.VMEM((B, D, _HWB), jnp.float32),
            pltpu.VMEM((B, _HWB), jnp.float32),
            pltpu.VMEM((_TJ, D, _HWB), jnp.float32),
        ],
    )
    out_shapes = (
        jax.ShapeDtypeStruct((B, HW), jnp.bool_),
        jax.ShapeDtypeStruct((B, HW), jnp.bool_),
        jax.ShapeDtypeStruct((B, HW), jnp.bool_),
        jax.ShapeDtypeStruct((B, HW), jnp.float32),
        jax.ShapeDtypeStruct((B, HW), jnp.int32),
    )
    m0, m1, m2, dmin, pcls = pl.pallas_call(
        _body,
        grid_spec=grid_spec,
        out_shape=out_shapes,
        compiler_params=pltpu.CompilerParams(
            dimension_semantics=("parallel", "arbitrary"),
        ),
    )(template_classes, frame_embeddings, templates)
    return m0, m1, m2, dmin, pcls
